# baseline (speedup 1.0000x reference)
import jax
import jax.numpy as jnp
from jax import lax
from jax.experimental import pallas as pl
from jax.experimental.pallas import tpu as pltpu
from jax.experimental.pallas import tpu_sc as plsc

BATCH = 16384
HIST_LEN = 50
D = 64
N = BATCH * HIST_LEN
NC, NS = 2, 16
NW = NC * NS
C = 128
CPW = N // (NW * C)
NBUF = 8
NGRP = CPW // NBUF
NWBUF = 2
BB = BATCH // C


def _emb_body(idx_hbm, table_hbm, out_hbm, idx_v, rows, trows, gsem, wsem):
    wid = lax.axis_index("s") * NC + lax.axis_index("c")
    q0 = wid * CPW
    pltpu.sync_copy(idx_hbm.at[pl.ds(q0, CPW)], idx_v)

    lane = lax.iota(jnp.int32, 16)

    def fire_gather(j, b):
        pltpu.async_copy(table_hbm.at[idx_v.at[j]], rows[b], gsem[b])

    def wait_gather(b):
        pltpu.make_async_copy(table_hbm.at[idx_v.at[0]], rows[b], gsem[b]).wait()

    def out_slice(j, t):
        q = q0 + j
        h = q // BB
        bb = q % BB
        return out_hbm.at[h, :, pl.ds(bb * C, C)]

    def fire_write(j, t):
        pltpu.async_copy(trows[t], out_slice(j, t), wsem[t])

    def wait_write(t):
        pltpu.make_async_copy(trows[t], out_hbm.at[0, :, pl.ds(0, C)], wsem[t]).wait()

    def transpose_block(b, t):
        def body(d, carry):
            dcol = jnp.full((16,), 0, jnp.int32) + d
            for k in range(C // 16):
                g = plsc.load_gather(rows[b], [lane + (16 * k), dcol])
                trows[t][d, pl.ds(16 * k, 16)] = g
            return carry

        lax.fori_loop(0, D, body, 0)

    def step(j, b, do_wait_write, do_fire):
        t = b % NWBUF
        wait_gather(b)
        if do_wait_write:
            wait_write(t)
        transpose_block(b, t)
        fire_write(j, t)
        if do_fire:
            bf = (b + NBUF - 1) % NBUF
            fire_gather(j + NBUF - 1, bf)

    for b in range(NBUF - 1):
        fire_gather(b, b)
    for b in range(NBUF):
        step(b, b, do_wait_write=(b >= NWBUF), do_fire=True)

    def group(g, carry):
        for b in range(NBUF):
            step(g * NBUF + b, b, do_wait_write=True, do_fire=True)
        return carry

    lax.fori_loop(1, NGRP - 1, group, 0)

    j0 = (NGRP - 1) * NBUF
    for b in range(NBUF):
        step(j0 + b, b, do_wait_write=True, do_fire=(b == 0))

    for t in range(NWBUF):
        wait_write(t)


@jax.jit
def kernel(traj, emb_weight):
    idx = traj.astype(jnp.int32).T.reshape(N // C, C)
    out = pl.kernel(
        _emb_body,
        out_type=jax.ShapeDtypeStruct((HIST_LEN, D, BATCH), jnp.float32),
        mesh=plsc.VectorSubcoreMesh(core_axis_name="c", subcore_axis_name="s"),
        compiler_params=pltpu.CompilerParams(
            use_tc_tiling_on_sc=False, needs_layout_passes=False
        ),
        scratch_types=[
            pltpu.VMEM((CPW, C), jnp.int32),
            [pltpu.VMEM((C, D), jnp.float32) for _ in range(NBUF)],
            [pltpu.VMEM((D, C), jnp.float32) for _ in range(NWBUF)],
            [pltpu.SemaphoreType.DMA for _ in range(NBUF)],
            [pltpu.SemaphoreType.DMA for _ in range(NWBUF)],
        ],
    )(idx, emb_weight)
    return out.transpose(2, 0, 1)


# speedup vs baseline: 1.6167x; 1.6167x over previous
import jax
import jax.numpy as jnp
from jax import lax
from jax.experimental import pallas as pl
from jax.experimental.pallas import tpu as pltpu
from jax.experimental.pallas import tpu_sc as plsc

BATCH = 16384
HIST_LEN = 50
D = 64
N = BATCH * HIST_LEN
NC, NS = 2, 16
NW = NC * NS
C = 128
CPW = N // (NW * C)
NBUF = 4
NGRP = CPW // NBUF
NWBUF = 2
BB = BATCH // C


def _emb_body(idx_hbm, table_hbm, out_hbm, idx_v, rows, idxg, parv, trows,
              gsem, wsem):
    wid = lax.axis_index("s") * NC + lax.axis_index("c")
    q0 = wid * CPW
    pltpu.sync_copy(idx_hbm.at[pl.ds(q0, CPW)], idx_v)

    lane = lax.iota(jnp.int32, 16)
    rowk = [lane + 16 * k for k in range(C // 16)]

    def fire_gather(j, b):
        for k in range(C // 16):
            v = idx_v[j, pl.ds(16 * k, 16)]
            idxg[b][pl.ds(16 * k, 16)] = v >> 1
            parv[b][pl.ds(16 * k, 16)] = (v & 1) << 6
        pltpu.async_copy(table_hbm.at[idxg[b]], rows[b], gsem[b])

    def wait_gather(b):
        pltpu.make_async_copy(table_hbm.at[idxg[b]], rows[b], gsem[b]).wait()

    def fire_write(j, t):
        q = q0 + j
        pltpu.async_copy(trows[t], out_hbm.at[q // BB, :, q % BB], wsem[t])

    def wait_write(t):
        pltpu.make_async_copy(trows[t], out_hbm.at[0, :, 0], wsem[t]).wait()

    def transpose_block(b, t):
        par64 = [parv[b][pl.ds(16 * k, 16)] for k in range(C // 16)]

        @plsc.parallel_loop(0, D, 1)
        def body(d):
            i = d >> 3
            off = (d & 7) * 128
            for k in range(C // 16):
                g = plsc.load_gather(rows[b], [rowk[k], par64[k] + d])
                trows[t][i, pl.ds(off + 16 * k, 16)] = g

    def step(j, b, do_wait_write, do_fire):
        t = b % NWBUF
        wait_gather(b)
        if do_wait_write:
            wait_write(t)
        transpose_block(b, t)
        fire_write(j, t)
        if do_fire:
            fire_gather(j + NBUF - 1, (b + NBUF - 1) % NBUF)

    for b in range(NBUF - 1):
        fire_gather(b, b)
    for b in range(NBUF):
        step(b, b, do_wait_write=(b >= NWBUF), do_fire=True)

    def group(g, carry):
        for b in range(NBUF):
            step(g * NBUF + b, b, do_wait_write=True, do_fire=True)
        return carry

    lax.fori_loop(1, NGRP - 1, group, 0)

    j0 = (NGRP - 1) * NBUF
    for b in range(NBUF):
        step(j0 + b, b, do_wait_write=True, do_fire=(b == 0))

    for t in range(NWBUF):
        wait_write(t)


@jax.jit
def kernel(traj, emb_weight):
    idx = traj.astype(jnp.int32).T.reshape(N // C, C)
    tbl = emb_weight.reshape(1000000 // 2, 128)
    out = pl.kernel(
        _emb_body,
        out_type=jax.ShapeDtypeStruct((HIST_LEN, D // 8, BB, 8 * C), jnp.float32),
        mesh=plsc.VectorSubcoreMesh(core_axis_name="c", subcore_axis_name="s"),
        compiler_params=pltpu.CompilerParams(
            use_tc_tiling_on_sc=False,
            needs_layout_passes=False,
            disable_bounds_checks=True,
        ),
        scratch_types=[
            pltpu.VMEM((CPW, C), jnp.int32),
            [pltpu.VMEM((C, 128), jnp.float32) for _ in range(NBUF)],
            [pltpu.VMEM((C,), jnp.int32) for _ in range(NBUF)],
            [pltpu.VMEM((C,), jnp.int32) for _ in range(NBUF)],
            [pltpu.VMEM((D // 8, 8 * C), jnp.float32) for _ in range(NWBUF)],
            [pltpu.SemaphoreType.DMA for _ in range(NBUF)],
            [pltpu.SemaphoreType.DMA for _ in range(NWBUF)],
        ],
    )(idx, tbl)
    out = out.reshape(HIST_LEN, D // 8, BB, 8, C)
    out = out.transpose(0, 1, 3, 2, 4).reshape(HIST_LEN, D, BATCH)
    return out.transpose(2, 0, 1)
